# baseline (speedup 1.0000x reference)
import functools

import jax
import jax.numpy as jnp
from jax import lax
from jax.experimental import pallas as pl
from jax.experimental.pallas import tpu as pltpu
from jax.experimental.pallas import tpu_sc as plsc

FEATURE_NUM = 26
VOCAB = 100000
EMBED_DIM = 32
BATCH = 16384

_L = 16
_NC = 2
_NS = 16
_NW = _NC * _NS

_BPW = BATCH // _NW
_CHUNK = 128
_CPF = _BPW // _CHUNK
_NJ = FEATURE_NUM * _CPF


def _body(x_hbm, table_hbm, out_hbm, xv, idx_v, rows0, rows1,
          gsem0, gsem1, wsem0, wsem1):
    wid = lax.axis_index("s") * _NC + lax.axis_index("c")
    b0 = wid * _BPW

    pltpu.sync_copy(x_hbm.at[pl.ds(b0, _BPW)], xv)

    lanes = lax.iota(jnp.int32, _L)

    def compute_chunk(j, _):
        f = lax.div(j, _CPF)
        fvec = jnp.full((_L,), 0, jnp.int32) + f
        for t in range(_CHUNK // _L):
            bvec = lax.rem(j, _CPF) * _CHUNK + t * _L + lanes
            idx_v[j, pl.ds(t * _L, _L)] = plsc.load_gather(xv, [bvec, fvec])
        return 0

    lax.fori_loop(0, _NJ, compute_chunk, 0)

    def fire(f, rows, gsem):
        for k in range(_CPF):
            pltpu.make_async_copy(
                table_hbm.at[f].at[idx_v.at[f * _CPF + k]],
                rows.at[pl.ds(k * _CHUNK, _CHUNK)],
                gsem,
            ).start()

    def drain_gathers(f, rows, gsem):
        for k in range(_CPF):
            pltpu.make_async_copy(
                table_hbm.at[f].at[idx_v.at[f * _CPF + k]],
                rows.at[pl.ds(k * _CHUNK, _CHUNK)],
                gsem,
            ).wait()

    def wb(f, rows, wsem):
        pltpu.make_async_copy(
            rows, out_hbm.at[pl.ds(b0, _BPW), f], wsem
        ).start()

    def wb_wait(rows, wsem):
        pltpu.make_async_copy(
            rows, out_hbm.at[pl.ds(b0, _BPW), 0], wsem
        ).wait()

    bufs = (rows0, rows1)
    gsems = (gsem0, gsem1)
    wsems = (wsem0, wsem1)

    fire(0, rows0, gsem0)
    for f in range(FEATURE_NUM):
        s = f % 2
        if f >= 1:
            wb_wait(bufs[1 - s], wsems[1 - s])
        if f + 1 < FEATURE_NUM:
            fire(f + 1, bufs[1 - s], gsems[1 - s])
        drain_gathers(f, bufs[s], gsems[s])
        wb(f, bufs[s], wsems[s])
    wb_wait(bufs[(FEATURE_NUM - 1) % 2], wsems[(FEATURE_NUM - 1) % 2])


@jax.jit
def _run(x, tables):
    kfn = pl.kernel(
        _body,
        mesh=plsc.VectorSubcoreMesh(core_axis_name="c", subcore_axis_name="s"),
        out_type=jax.ShapeDtypeStruct((BATCH, FEATURE_NUM, EMBED_DIM), jnp.float32),
        scratch_types=[
            pltpu.VMEM((_BPW, FEATURE_NUM), jnp.int32),
            pltpu.VMEM((_NJ, _CHUNK), jnp.int32),
            pltpu.VMEM((_BPW, EMBED_DIM), jnp.float32),
            pltpu.VMEM((_BPW, EMBED_DIM), jnp.float32),
            pltpu.SemaphoreType.DMA,
            pltpu.SemaphoreType.DMA,
            pltpu.SemaphoreType.DMA,
            pltpu.SemaphoreType.DMA,
        ],
        compiler_params=pltpu.CompilerParams(
            use_tc_tiling_on_sc=False, needs_layout_passes=False
        ),
    )
    return kfn(x, tables)


def kernel(x, tables):
    return _run(x.astype(jnp.int32), tables)
